# baseline (speedup 1.0000x reference)
import functools

import jax
import jax.numpy as jnp
from jax import lax
from jax.experimental import pallas as pl
from jax.experimental.pallas import tpu as pltpu
from jax.experimental.pallas import tpu_sc as plsc

L_SPINS = 20
BATCH = 16384
DELTA = 1e-15

NUM_CORES = 2
NUM_SUBCORES = 16
NW = NUM_CORES * NUM_SUBCORES
BW = BATCH // NW
CHUNK = 128
NCHUNK = BW // CHUNK
VPC = CHUNK // 16

_LN2 = 0.6931472
_SQRT2 = 1.4142135
_PI = 3.14159265358979


def _log_abs(v):
    x = jnp.abs(v + jnp.float32(DELTA))
    bits = lax.bitcast_convert_type(x, jnp.int32)
    e = (bits >> 23) - 127
    m = lax.bitcast_convert_type((bits & 0x007FFFFF) | 0x3F800000, jnp.float32)
    big = m > jnp.float32(_SQRT2)
    m = jnp.where(big, m * jnp.float32(0.5), m)
    e = jnp.where(big, e + 1, e)
    ef = e.astype(jnp.float32)
    t = (m - jnp.float32(1.0)) / (m + jnp.float32(1.0))
    t2 = t * t
    p = t * (jnp.float32(2.0)
             + t2 * (jnp.float32(2 / 3)
                     + t2 * (jnp.float32(2 / 5) + t2 * jnp.float32(2 / 7))))
    return ef * jnp.float32(_LN2) + p


def _tc_index_kernel(s3):
    def body(s_ref, idx_ref):
        acc = s_ref[0]
        for l in range(1, L_SPINS):
            acc = acc + (s_ref[l] << l)
        idx_ref[...] = acc

    return pl.pallas_call(
        body,
        out_shape=jax.ShapeDtypeStruct((BATCH // 128, 128), jnp.int32),
    )(s3)


def _sc_kernel(idx, table):
    mesh = plsc.VectorSubcoreMesh(core_axis_name="c", subcore_axis_name="s")

    @functools.partial(
        pl.kernel,
        out_type=jax.ShapeDtypeStruct((2, BATCH), jnp.float32),
        mesh=mesh,
        scratch_types=[
            pltpu.VMEM((NCHUNK, CHUNK), jnp.int32),
            pltpu.VMEM((NCHUNK, CHUNK), jnp.float32),
            pltpu.VMEM((2, BW), jnp.float32),
            pltpu.SemaphoreType.DMA,
            pltpu.SemaphoreType.DMA,
        ],
    )
    def k(idx_hbm, t_hbm, out_hbm, idx_v, vals_v, out_v, sem_i, sem_g):
        wid = lax.axis_index("s") * NUM_CORES + lax.axis_index("c")
        base = wid * BW

        idx_copies = [
            pltpu.make_async_copy(
                idx_hbm.at[pl.ds(base + j * CHUNK, CHUNK)], idx_v.at[j], sem_i)
            for j in range(NCHUNK)
        ]
        for c in idx_copies:
            c.start()

        gathers = []
        for j in range(NCHUNK):
            idx_copies[j].wait()
            g = pltpu.make_async_copy(t_hbm.at[idx_v.at[j]], vals_v.at[j], sem_g)
            g.start()
            gathers.append(g)

        for j in range(NCHUNK):
            gathers[j].wait()

            def elem(kk, carry, j=j):
                off = j * CHUNK + kk * 16
                v = vals_v[j, pl.ds(kk * 16, 16)]
                vbits = lax.bitcast_convert_type(v, jnp.int32)
                out_v[0, pl.ds(off, 16)] = _log_abs(v)
                out_v[1, pl.ds(off, 16)] = jnp.where(
                    vbits < 0, jnp.float32(_PI), jnp.float32(0.0))
                return carry
            lax.fori_loop(0, VPC, elem, 0)

        pltpu.sync_copy(out_v, out_hbm.at[:, pl.ds(base, BW)])

    return k(idx, table)


def kernel(s, kernel):
    s3 = s.reshape(L_SPINS, BATCH // 128, 128)
    idx = _tc_index_kernel(s3).reshape(BATCH)
    out = _sc_kernel(idx, kernel)
    return lax.complex(out[0], out[1])


# speedup vs baseline: 1.0464x; 1.0464x over previous
import functools

import jax
import jax.numpy as jnp
from jax import lax
from jax.experimental import pallas as pl
from jax.experimental.pallas import tpu as pltpu
from jax.experimental.pallas import tpu_sc as plsc

L_SPINS = 20
BATCH = 16384
DELTA = 1e-15

NUM_CORES = 2
NUM_SUBCORES = 16
NW = NUM_CORES * NUM_SUBCORES
BW = BATCH // NW
CHUNK = 128
NCHUNK = BW // CHUNK
VPC = CHUNK // 16

_LN2 = 0.6931472
_SQRT2 = 1.4142135
_PI = 3.14159265358979


def _log_abs(v):
    x = jnp.abs(v + jnp.float32(DELTA))
    bits = lax.bitcast_convert_type(x, jnp.int32)
    e = (bits >> 23) - 127
    m = lax.bitcast_convert_type((bits & 0x007FFFFF) | 0x3F800000, jnp.float32)
    big = m > jnp.float32(_SQRT2)
    m = jnp.where(big, m * jnp.float32(0.5), m)
    e = jnp.where(big, e + 1, e)
    ef = e.astype(jnp.float32)
    t = (m - jnp.float32(1.0)) / (m + jnp.float32(1.0))
    t2 = t * t
    p = t * (jnp.float32(2.0)
             + t2 * (jnp.float32(2 / 3)
                     + t2 * (jnp.float32(2 / 5) + t2 * jnp.float32(2 / 7))))
    return ef * jnp.float32(_LN2) + p


def _sc_kernel(s, table):
    mesh = plsc.VectorSubcoreMesh(core_axis_name="c", subcore_axis_name="s")

    @functools.partial(
        pl.kernel,
        out_type=jax.ShapeDtypeStruct((2, BATCH), jnp.float32),
        mesh=mesh,
        scratch_types=[
            pltpu.VMEM((L_SPINS, BW), jnp.int32),
            pltpu.VMEM((NCHUNK, CHUNK), jnp.int32),
            pltpu.VMEM((NCHUNK, CHUNK), jnp.float32),
            pltpu.VMEM((2, BW), jnp.float32),
            pltpu.SemaphoreType.DMA,
            pltpu.SemaphoreType.DMA,
        ],
    )
    def k(s_hbm, t_hbm, out_hbm, s_v, idx_v, vals_v, out_v, sem_s, sem_g):
        wid = lax.axis_index("s") * NUM_CORES + lax.axis_index("c")
        base = wid * BW

        s_copies = []
        for j in range(NCHUNK):
            c = pltpu.make_async_copy(
                s_hbm.at[:, pl.ds(base + j * CHUNK, CHUNK)],
                s_v.at[:, pl.ds(j * CHUNK, CHUNK)], sem_s)
            c.start()
            s_copies.append(c)

        gathers = []
        for j in range(NCHUNK):
            s_copies[j].wait()

            def build(kk, carry, j=j):
                off = j * CHUNK + kk * 16
                acc = s_v[0, pl.ds(off, 16)]
                for l in range(1, L_SPINS):
                    acc = acc + (s_v[l, pl.ds(off, 16)] << l)
                idx_v[j, pl.ds(kk * 16, 16)] = acc
                return carry
            lax.fori_loop(0, VPC, build, 0)
            g = pltpu.make_async_copy(t_hbm.at[idx_v.at[j]], vals_v.at[j], sem_g)
            g.start()
            gathers.append(g)

        for j in range(NCHUNK):
            gathers[j].wait()

            def elem(kk, carry, j=j):
                off = j * CHUNK + kk * 16
                v = vals_v[j, pl.ds(kk * 16, 16)]
                vbits = lax.bitcast_convert_type(v, jnp.int32)
                out_v[0, pl.ds(off, 16)] = _log_abs(v)
                out_v[1, pl.ds(off, 16)] = jnp.where(
                    vbits < 0, jnp.float32(_PI), jnp.float32(0.0))
                return carry
            lax.fori_loop(0, VPC, elem, 0)

        pltpu.sync_copy(out_v, out_hbm.at[:, pl.ds(base, BW)])

    return k(s, table)


def kernel(s, kernel):
    out = _sc_kernel(s, kernel)
    return lax.complex(out[0], out[1])
